# baseline (speedup 1.0000x reference)
import functools

import jax
import jax.numpy as jnp
from jax import lax
from jax.experimental import pallas as pl
from jax.experimental.pallas import tpu as pltpu
from jax.experimental.pallas import tpu_sc as plsc

BATCH = 16384
LATENT_D = 32
HIST = 50
NC = 2
NS = 16
NW = NC * NS
BPW = BATCH // NW
CHUNK = 128
NCH = BPW // CHUNK
L = 16
GROUPS = BPW // L
SLAB = 128


def _rsqrt(x):
    i = plsc.bitcast(x, jnp.int32)
    y = plsc.bitcast(jnp.int32(0x5F3759DF) - (i >> 1), jnp.float32)
    xh = x * -0.5
    y = y * (1.5 + xh * y * y)
    y = y * (1.5 + xh * y * y)
    y = y * (1.5 + xh * y * y)
    return y


@functools.partial(
    pl.kernel,
    mesh=plsc.VectorSubcoreMesh(core_axis_name="c", subcore_axis_name="s"),
    compiler_params=pltpu.CompilerParams(needs_layout_passes=False),
    out_type=(
        jax.ShapeDtypeStruct((BATCH,), jnp.float32),
        jax.ShapeDtypeStruct((BATCH,), jnp.float32),
    ),
    scratch_types=[
        pltpu.VMEM((NCH, CHUNK), jnp.int32),
        pltpu.VMEM((NCH, CHUNK), jnp.int32),
        pltpu.VMEM((NCH, CHUNK), jnp.int32),
        pltpu.VMEM((NCH, CHUNK), jnp.int32),
        pltpu.VMEM((BPW,), jnp.int32),
        pltpu.VMEM((BPW,), jnp.int32),
        pltpu.VMEM((BPW * HIST,), jnp.float32),
        pltpu.VMEM((CHUNK, SLAB), jnp.float32),
        pltpu.VMEM((CHUNK, SLAB), jnp.float32),
        pltpu.VMEM((BPW,), jnp.float32),
        pltpu.VMEM((BPW,), jnp.float32),
        pltpu.VMEM((BPW,), jnp.float32),
        pltpu.VMEM((BPW,), jnp.float32),
        pltpu.VMEM((L,), jnp.float32),
        pltpu.VMEM((L,), jnp.float32),
        pltpu.VMEM((L,), jnp.float32),
        pltpu.VMEM((L,), jnp.float32),
        pltpu.VMEM((L,), jnp.float32),
        pltpu.VMEM((BPW,), jnp.float32),
        pltpu.VMEM((BPW,), jnp.float32),
        pltpu.SemaphoreType.DMA,
    ],
)
def _ex2vec_sc(uidx_h, iidx_h, su_h, si_h, qu_h, qi_h, r_h, eu_h, ei_h,
               ulamb_h, ubias_h, ibias_h,
               cvec_h, glvec_h, avec_h, bvec_h, gvec_h,
               oint_h, odist_h,
               uidx_v, iidx_v, su_v, si_v, qu_v, qi_v, rchunk, uslab, islab,
               ulamb_v, ubias_v, ibias_v, bd_v,
               cvec_v, glvec_v, avec_v, bvec_v, gvec_v,
               oint_v, odist_v, sem):
    cid = lax.axis_index("c")
    sid = lax.axis_index("s")
    wid = sid * NC + cid
    base = wid * BPW

    pltpu.sync_copy(uidx_h.at[wid], uidx_v)
    pltpu.sync_copy(iidx_h.at[wid], iidx_v)
    pltpu.sync_copy(su_h.at[wid], su_v)
    pltpu.sync_copy(si_h.at[wid], si_v)
    pltpu.sync_copy(qu_h.at[pl.ds(base, BPW)], qu_v)
    pltpu.sync_copy(qi_h.at[pl.ds(base, BPW)], qi_v)
    pltpu.sync_copy(r_h.at[pl.ds(base * HIST, BPW * HIST)], rchunk)
    pltpu.sync_copy(cvec_h, cvec_v)
    pltpu.sync_copy(glvec_h, glvec_v)
    pltpu.sync_copy(avec_h, avec_v)
    pltpu.sync_copy(bvec_h, bvec_v)
    pltpu.sync_copy(gvec_h, gvec_v)

    bias_copies = []
    for j in range(NCH):
        ssl = pl.ds(j * CHUNK, CHUNK)
        bias_copies.append(
            pltpu.async_copy(ulamb_h.at[uidx_v.at[j]], ulamb_v.at[ssl], sem))
        bias_copies.append(
            pltpu.async_copy(ubias_h.at[uidx_v.at[j]], ubias_v.at[ssl], sem))
        bias_copies.append(
            pltpu.async_copy(ibias_h.at[iidx_v.at[j]], ibias_v.at[ssl], sem))
    for c in bias_copies:
        c.wait()

    iota = lax.iota(jnp.int32, L)

    def chunk_body(j, carry):
        cu = pltpu.async_copy(eu_h.at[su_v.at[j]], uslab, sem)
        ci = pltpu.async_copy(ei_h.at[si_v.at[j]], islab, sem)
        cu.wait()
        ci.wait()

        def grp(g, c2):
            rows = iota + g * L
            ebase = j * CHUNK + g * L
            qu = qu_v[pl.ds(ebase, L)]
            qi = qi_v[pl.ds(ebase, L)]
            ucol = qu
            icol = qi
            acc = jnp.zeros((L,), jnp.float32)
            for d in range(LATENT_D):
                u = plsc.load_gather(uslab, [rows, ucol + d])
                it = plsc.load_gather(islab, [rows, icol + d])
                acc = acc + jnp.abs(it - u)
            bd_v[pl.ds(ebase, L)] = acc
            return c2

        lax.fori_loop(0, CHUNK // L, grp, 0)
        return carry

    lax.fori_loop(0, NCH, chunk_body, 0)

    C = cvec_v[...]
    GL = glvec_v[...]
    AV = avec_v[...]
    BV = bvec_v[...]
    GV = gvec_v[...]

    def final_body(g, carry):
        gbase = g * L
        rows_h = (iota + gbase) * HIST
        bl = jnp.zeros((L,), jnp.float32)
        for h in range(HIST):
            r = plsc.load_gather(rchunk, [rows_h + h])
            m = r > 0.0
            dt = jnp.where(m, r, 0.0) + C
            y = _rsqrt(dt)
            bl = bl + jnp.where(m, y, 0.0)
        bd = bd_v[pl.ds(gbase, L)]
        ul = ulamb_v[pl.ds(gbase, L)]
        lamb = GL + jnp.clip(ul, 0.1, 10.0)
        act = jnp.minimum(bl * lamb, bd)
        dist = bd - act
        ub = ubias_v[pl.ds(gbase, L)]
        ib = ibias_v[pl.ds(gbase, L)]
        I = AV * dist + BV * dist * dist + GV + ub + ib
        interest = 1.0 / (1.0 + jnp.exp(-I))
        oint_v[pl.ds(gbase, L)] = interest
        odist_v[pl.ds(gbase, L)] = dist
        return carry

    lax.fori_loop(0, GROUPS, final_body, 0)
    pltpu.sync_copy(oint_v, oint_h.at[pl.ds(base, BPW)])
    pltpu.sync_copy(odist_v, odist_h.at[pl.ds(base, BPW)])


def kernel(user_indices, item_indices, r_interval, embedding_user, embedding_item,
           user_lamb, user_bias, item_bias, global_lamb, alpha, beta, gamma, cutoff):
    f32 = jnp.float32
    uidx = user_indices.astype(jnp.int32)
    iidx = item_indices.astype(jnp.int32)
    uidx3 = uidx.reshape(NW, NCH, CHUNK)
    iidx3 = iidx.reshape(NW, NCH, CHUNK)
    su3 = (uidx >> 2).reshape(NW, NCH, CHUNK)
    si3 = (iidx >> 2).reshape(NW, NCH, CHUNK)
    qu = (uidx & 3) * LATENT_D
    qi = (iidx & 3) * LATENT_D
    r_flat = r_interval.reshape(-1)
    eu_slab = embedding_user.reshape(-1, SLAB)
    ei_slab = embedding_item.reshape(-1, SLAB)
    ulamb_flat = user_lamb.reshape(-1)
    ubias_flat = user_bias.reshape(-1)
    ibias_flat = item_bias.reshape(-1)

    def bc(v):
        return jnp.full((L,), v, f32)

    cvec = bc(jnp.clip(cutoff.astype(f32), 0.1, 100.0))
    glvec = bc(jnp.clip(global_lamb.astype(f32), 0.01, 10.0))
    avec = bc(alpha)
    bvec = bc(beta)
    gvec = bc(gamma)
    interest, distance = _ex2vec_sc(
        uidx3, iidx3, su3, si3, qu, qi, r_flat, eu_slab, ei_slab,
        ulamb_flat, ubias_flat, ibias_flat, cvec, glvec, avec, bvec, gvec)
    return interest, distance
